# baseline (speedup 1.0000x reference)
import jax
import jax.numpy as jnp
from jax import lax
from jax.experimental import pallas as pl
from jax.experimental.pallas import tpu as pltpu

NI = 8192
NBAGS = 256
BLK = 128
NBLK = NI // BLK


def _k1_body(x_ref, w1_ref, b1_ref, w2_ref, b2_ref, a1w_ref, a1b_ref,
             a1ow_ref, a1ob_ref, lab_ref, u_ref, sv_ref, s1, sr, sc2, s2,
             sx3, sc1):
    step = pl.program_id(0)

    @pl.when(step == 0)
    def _():
        s1[:, 13, :, :] = jnp.zeros((13, 16, 128), jnp.float32)
        for d in range(3):
            sx3[pl.ds(d * 32 + 28, 4), :] = jnp.zeros((4, 26 * 128),
                                                      jnp.float32)

    for d in range(3):
        for y in range(26):
            sx3[pl.ds(d * 32, 28), pl.ds(y * 128, 128)] = x_ref[0, d + y]

    c1 = jnp.dot(w1_ref[...], sx3[...],
                 preferred_element_type=jnp.float32)
    sc1[...] = c1

    for yp in range(13):
        c0 = sc1[:, pl.ds((2 * yp) * 128, 128)]
        c1r = sc1[:, pl.ds((2 * yp + 1) * 128, 128)]
        m = jnp.maximum(jnp.maximum(c0, c1r) + b1_ref[...], 0.0)
        px = jnp.max(m.reshape(13, 2, 16, 128), axis=1)
        s1[yp, 0:13, :, :] = px

    def asm(j, _):
        yo = j // 3
        g = j - yo * 3
        for dy in range(3):
            for dxg in range(6):
                t = s1[yo + dy, g * 4 + dxg, :, :]
                sr[pl.ds((dy * 6 + dxg) * 16, 16), pl.ds(j * 128, 128)] = t
        return 0

    lax.fori_loop(0, 33, asm, 0)
    r = sr[...]
    c = jnp.dot(w2_ref[...], r, preferred_element_type=jnp.float32)
    c = jnp.maximum(c + b2_ref[...], 0.0)
    sc2[...] = c

    def fill2(j, _):
        yo = j // 3
        g = j - yo * 3
        p = sc2[:, pl.ds(j * 128, 128)]
        s2[yo, pl.ds(g * 4, 4)] = p.reshape(4, 32, 128)
        return 0

    lax.fori_loop(0, 33, fill2, 0)

    pieces = []
    for yo2 in range(5):
        r0 = s2[2 * yo2]
        r1 = s2[2 * yo2 + 1]
        m = jnp.maximum(r0, r1)[0:10]
        p = jnp.max(m.reshape(5, 2, 32, 128), axis=1)
        pieces.append(p.reshape(160, 128))
    emb = jnp.concatenate(pieces, axis=0)

    t1 = jnp.tanh(jnp.dot(a1w_ref[...], emb,
                          preferred_element_type=jnp.float32) + a1b_ref[...])
    lg = jnp.dot(a1ow_ref[...], t1,
                 preferred_element_type=jnp.float32) + a1ob_ref[...]
    e = jnp.exp(jax.nn.sigmoid(lg))

    seg = lax.broadcasted_iota(jnp.int32, (NBAGS, 128), 0)
    ohw = jnp.where(lab_ref[0] == seg, e, 0.0)
    contrib = lax.dot_general(ohw, emb, (((1,), (1,)), ((), ())),
                              preferred_element_type=jnp.float32)

    @pl.when(step == 0)
    def _():
        u_ref[...] = jnp.zeros((NBAGS, 800), jnp.float32)
        sv_ref[...] = jnp.zeros((8, 128), jnp.float32)

    u_ref[...] += contrib
    sv_ref[0:1, :] += e


def _k3_body(u_ref, sv_ref, a2w_ref, a2b_ref, a2ow_ref, a2ob_ref,
             cw_ref, cb_ref, cow_ref, cob_ref, o_ref):
    u = u_ref[...]
    inv_s = 1.0 / jnp.sum(sv_ref[...])
    t3 = jnp.tanh(jnp.dot(u, a2w_ref[...],
                          preferred_element_type=jnp.float32) * inv_s
                  + a2b_ref[...])
    l3 = jax.nn.sigmoid(jnp.dot(t3, a2ow_ref[...],
                                preferred_element_type=jnp.float32)
                        + a2ob_ref[...])
    w3 = jnp.exp(l3)
    z = jnp.sum(w3)
    outer = lax.dot_general(w3, u, (((0,), (0,)), ((), ())),
                            preferred_element_type=jnp.float32)
    outer = outer * (inv_s / z)
    p1 = jnp.dot(outer, cw_ref[...],
                 preferred_element_type=jnp.float32) + cb_ref[...]
    p2 = jnp.dot(p1, cow_ref[...],
                 preferred_element_type=jnp.float32) + cob_ref[...]
    o_ref[...] = jnp.broadcast_to(jax.nn.sigmoid(p2), (8, 128))


def kernel(x, first_lab, conv1_w, conv1_b, conv2_w, conv2_b, a1_w, a1_b,
           a1o_w, a1o_b, a2_w, a2_b, a2o_w, a2o_b, c_w, c_b, co_w, co_b):
    xt = jnp.transpose(x.reshape(NBLK, BLK, 28, 28), (0, 2, 3, 1))
    t1 = conv1_w[:, :, 0, :]
    xo = jnp.arange(26)[:, None, None, None]
    cc = jnp.arange(16)[None, :, None, None]
    dd = jnp.arange(3)[None, None, :, None]
    xi = jnp.arange(32)[None, None, None, :]
    rel = xi - xo
    w1s = jnp.where((rel >= 0) & (rel <= 2) & (xi < 28),
                    t1[dd, jnp.clip(rel, 0, 2), cc], 0.0).reshape(416, 96)
    b1s = jnp.tile(conv1_b.reshape(1, 16), (26, 1)).reshape(416, 1)
    x4 = jnp.arange(4)[:, None, None, None, None]
    co = jnp.arange(32)[None, :, None, None, None]
    dy = jnp.arange(3)[None, None, :, None, None]
    dxg = jnp.arange(6)[None, None, None, :, None]
    ci = jnp.arange(16)[None, None, None, None, :]
    rel2 = dxg - x4
    w2big = jnp.where((rel2 >= 0) & (rel2 <= 2),
                      conv2_w[dy, jnp.clip(rel2, 0, 2), ci, co],
                      0.0).reshape(128, 288)
    b2big = jnp.tile(conv2_b.reshape(1, 32), (4, 1)).reshape(128, 1)
    a1wt = a1_w.T
    a1bc = a1_b.reshape(64, 1)
    a1owt = a1o_w.reshape(1, 64)
    a1obc = a1o_b.reshape(1, 1)
    lab3 = first_lab.astype(jnp.int32).reshape(NBLK, 1, BLK)

    full = lambda shape: pl.BlockSpec(shape, lambda i: tuple(0 for _ in shape))
    u, sv = pl.pallas_call(
        _k1_body,
        grid=(NBLK,),
        in_specs=[
            pl.BlockSpec((1, 28, 28, BLK), lambda i: (i, 0, 0, 0)),
            full((416, 96)),
            full((416, 1)),
            full((128, 288)),
            full((128, 1)),
            full((64, 800)),
            full((64, 1)),
            full((1, 64)),
            full((1, 1)),
            pl.BlockSpec((1, 1, BLK), lambda i: (i, 0, 0)),
        ],
        out_specs=[full((NBAGS, 800)), full((8, 128))],
        out_shape=[jax.ShapeDtypeStruct((NBAGS, 800), jnp.float32),
                   jax.ShapeDtypeStruct((8, 128), jnp.float32)],
        scratch_shapes=[
            pltpu.VMEM((13, 14, 16, BLK), jnp.float32),
            pltpu.VMEM((288, 33 * BLK), jnp.float32),
            pltpu.VMEM((128, 33 * BLK), jnp.float32),
            pltpu.VMEM((11, 12, 32, BLK), jnp.float32),
            pltpu.VMEM((96, 26 * BLK), jnp.float32),
            pltpu.VMEM((416, 26 * BLK), jnp.float32),
        ],
    )(xt, w1s, b1s, w2big, b2big, a1wt, a1bc, a1owt, a1obc, lab3)

    out = pl.pallas_call(
        _k3_body,
        out_shape=jax.ShapeDtypeStruct((8, 128), jnp.float32),
    )(u, sv, a2_w, a2_b.reshape(1, 64), a2o_w, a2o_b.reshape(1, 1),
      c_w, c_b.reshape(1, 128), co_w, co_b.reshape(1, 1))
    return out[0:1, 0:1]


# speedup vs baseline: 3.6096x; 3.6096x over previous
import jax
import jax.numpy as jnp
from jax import lax
from jax.experimental import pallas as pl
from jax.experimental.pallas import tpu as pltpu

NI = 8192
NBAGS = 256
BLK = 256
NBLK = NI // BLK


def _k1_body(x_ref, w1_ref, b1_ref, w2_ref, b2_ref, a1w_ref, a1b_ref,
             a1ow_ref, a1ob_ref, lab_ref, u_ref, sv_ref, s1, sr, sc2, s2,
             sx3, sc1):
    step = pl.program_id(0)

    @pl.when(step == 0)
    def _():
        s1[:, 13, :, :] = jnp.zeros((13, 16, BLK), jnp.float32)
        for d in range(3):
            sx3[pl.ds(d * 32 + 28, 4), :] = jnp.zeros((4, 26 * BLK),
                                                      jnp.float32)

    for d in range(3):
        for y in range(26):
            sx3[pl.ds(d * 32, 28), pl.ds(y * BLK, BLK)] = x_ref[0, d + y]

    c1 = jnp.dot(w1_ref[...], sx3[...],
                 preferred_element_type=jnp.float32)
    sc1[...] = c1

    for yp in range(13):
        c0 = sc1[:, pl.ds((2 * yp) * BLK, BLK)]
        c1r = sc1[:, pl.ds((2 * yp + 1) * BLK, BLK)]
        m = jnp.maximum(jnp.maximum(c0, c1r) + b1_ref[...], 0.0)
        px = jnp.max(m.reshape(13, 2, 16, BLK), axis=1)
        s1[yp, 0:13, :, :] = px

    def asm(j, _):
        yo = j // 3
        g = j - yo * 3
        for dy in range(3):
            for dxg in range(6):
                t = s1[yo + dy, g * 4 + dxg, :, :]
                sr[pl.ds((dy * 6 + dxg) * 16, 16), pl.ds(j * BLK, BLK)] = t
        return 0

    lax.fori_loop(0, 33, asm, 0)
    r = sr[...]
    c = jnp.dot(w2_ref[...], r, preferred_element_type=jnp.float32)
    c = jnp.maximum(c + b2_ref[...], 0.0)
    sc2[...] = c

    def fill2(j, _):
        yo = j // 3
        g = j - yo * 3
        p = sc2[:, pl.ds(j * BLK, BLK)]
        s2[yo, pl.ds(g * 4, 4)] = p.reshape(4, 32, BLK)
        return 0

    lax.fori_loop(0, 33, fill2, 0)

    pieces = []
    for yo2 in range(5):
        r0 = s2[2 * yo2]
        r1 = s2[2 * yo2 + 1]
        m = jnp.maximum(r0, r1)[0:10]
        p = jnp.max(m.reshape(5, 2, 32, BLK), axis=1)
        pieces.append(p.reshape(160, BLK))
    emb = jnp.concatenate(pieces, axis=0)

    t1 = jnp.tanh(jnp.dot(a1w_ref[...], emb,
                          preferred_element_type=jnp.float32) + a1b_ref[...])
    lg = jnp.dot(a1ow_ref[...], t1,
                 preferred_element_type=jnp.float32) + a1ob_ref[...]
    e = jnp.exp(jax.nn.sigmoid(lg))

    seg = lax.broadcasted_iota(jnp.int32, (NBAGS, BLK), 0)
    ohw = jnp.where(lab_ref[0] == seg, e, 0.0)
    contrib = lax.dot_general(ohw, emb, (((1,), (1,)), ((), ())),
                              preferred_element_type=jnp.float32)

    @pl.when(step == 0)
    def _():
        u_ref[...] = jnp.zeros((NBAGS, 800), jnp.float32)
        sv_ref[...] = jnp.zeros((8, BLK), jnp.float32)

    u_ref[...] += contrib
    sv_ref[0:1, :] += e


def _k3_body(u_ref, sv_ref, a2w_ref, a2b_ref, a2ow_ref, a2ob_ref,
             cw_ref, cb_ref, cow_ref, cob_ref, o_ref):
    u = u_ref[...]
    inv_s = 1.0 / jnp.sum(sv_ref[...])
    t3 = jnp.tanh(jnp.dot(u, a2w_ref[...],
                          preferred_element_type=jnp.float32) * inv_s
                  + a2b_ref[...])
    l3 = jax.nn.sigmoid(jnp.dot(t3, a2ow_ref[...],
                                preferred_element_type=jnp.float32)
                        + a2ob_ref[...])
    w3 = jnp.exp(l3)
    z = jnp.sum(w3)
    outer = lax.dot_general(w3, u, (((0,), (0,)), ((), ())),
                            preferred_element_type=jnp.float32)
    outer = outer * (inv_s / z)
    p1 = jnp.dot(outer, cw_ref[...],
                 preferred_element_type=jnp.float32) + cb_ref[...]
    p2 = jnp.dot(p1, cow_ref[...],
                 preferred_element_type=jnp.float32) + cob_ref[...]
    o_ref[...] = jnp.broadcast_to(jax.nn.sigmoid(p2), (8, 128))


def kernel(x, first_lab, conv1_w, conv1_b, conv2_w, conv2_b, a1_w, a1_b,
           a1o_w, a1o_b, a2_w, a2_b, a2o_w, a2o_b, c_w, c_b, co_w, co_b):
    xt = jnp.transpose(x.reshape(NBLK, BLK, 28, 28), (0, 2, 3, 1))
    t1 = conv1_w[:, :, 0, :]
    w1s = jnp.stack(
        [jnp.transpose(jnp.pad(t1, ((0, 0), (xo, 29 - xo), (0, 0))),
                       (2, 0, 1)).reshape(16, 96)
         for xo in range(26)], axis=0).reshape(416, 96)
    b1s = jnp.tile(conv1_b.reshape(1, 16), (26, 1)).reshape(416, 1)
    t = jnp.transpose(conv2_w, (3, 0, 1, 2))
    w2big = jnp.stack(
        [jnp.pad(t, ((0, 0), (0, 0), (xi, 3 - xi), (0, 0))).reshape(32, 288)
         for xi in range(4)], axis=0).reshape(128, 288)
    b2big = jnp.tile(conv2_b.reshape(1, 32), (4, 1)).reshape(128, 1)
    a1wt = a1_w.T
    a1bc = a1_b.reshape(64, 1)
    a1owt = a1o_w.reshape(1, 64)
    a1obc = a1o_b.reshape(1, 1)
    lab3 = first_lab.astype(jnp.int32).reshape(NBLK, 1, BLK)

    full = lambda shape: pl.BlockSpec(shape, lambda i: tuple(0 for _ in shape))
    u, sv = pl.pallas_call(
        _k1_body,
        grid=(NBLK,),
        in_specs=[
            pl.BlockSpec((1, 28, 28, BLK), lambda i: (i, 0, 0, 0)),
            full((416, 96)),
            full((416, 1)),
            full((128, 288)),
            full((128, 1)),
            full((64, 800)),
            full((64, 1)),
            full((1, 64)),
            full((1, 1)),
            pl.BlockSpec((1, 1, BLK), lambda i: (i, 0, 0)),
        ],
        out_specs=[full((NBAGS, 800)), full((8, BLK))],
        out_shape=[jax.ShapeDtypeStruct((NBAGS, 800), jnp.float32),
                   jax.ShapeDtypeStruct((8, BLK), jnp.float32)],
        scratch_shapes=[
            pltpu.VMEM((13, 14, 16, BLK), jnp.float32),
            pltpu.VMEM((288, 33 * BLK), jnp.float32),
            pltpu.VMEM((128, 33 * BLK), jnp.float32),
            pltpu.VMEM((11, 12, 32, BLK), jnp.float32),
            pltpu.VMEM((96, 26 * BLK), jnp.float32),
            pltpu.VMEM((416, 26 * BLK), jnp.float32),
        ],
    )(xt, w1s, b1s, w2big, b2big, a1wt, a1bc, a1owt, a1obc, lab3)

    out = pl.pallas_call(
        _k3_body,
        out_shape=jax.ShapeDtypeStruct((8, 128), jnp.float32),
    )(u, sv, a2_w, a2_b.reshape(1, 64), a2o_w, a2o_b.reshape(1, 1),
      c_w, c_b.reshape(1, 128), co_w, co_b.reshape(1, 1))
    return out[0:1, 0:1]


# speedup vs baseline: 4.0103x; 1.1110x over previous
import jax
import jax.numpy as jnp
from jax import lax
from jax.experimental import pallas as pl
from jax.experimental.pallas import tpu as pltpu

NI = 8192
NBAGS = 256
BLK = 128
NBLK = NI // BLK


def _k1_body(x_ref, w1_ref, b1_ref, w2_ref, b2_ref, a1w_ref, a1b_ref,
             a1ow_ref, a1ob_ref, lab_ref, u_ref, sv_ref, s1, sr, sc2, s2,
             sx3, sc1):
    step = pl.program_id(0)

    @pl.when(step == 0)
    def _():
        s1[:, 13, :, :] = jnp.zeros((13, 16, 128), jnp.float32)
        for d in range(3):
            sx3[pl.ds(d * 32 + 28, 4), :] = jnp.zeros((4, 26 * 128),
                                                      jnp.float32)

    for d in range(3):
        for y in range(26):
            sx3[pl.ds(d * 32, 28), pl.ds(y * 128, 128)] = x_ref[0, d + y]

    c1 = jnp.dot(w1_ref[...], sx3[...],
                 preferred_element_type=jnp.float32)
    sc1[...] = c1

    for yp in range(13):
        c0 = sc1[:, pl.ds((2 * yp) * 128, 128)]
        c1r = sc1[:, pl.ds((2 * yp + 1) * 128, 128)]
        m = jnp.maximum(jnp.maximum(c0, c1r) + b1_ref[...], 0.0)
        px = jnp.max(m.reshape(13, 2, 16, 128), axis=1)
        s1[yp, 0:13, :, :] = px

    def asm(j, _):
        yo = j // 3
        g = j - yo * 3
        for dy in range(3):
            for dxg in range(6):
                t = s1[yo + dy, g * 4 + dxg, :, :]
                sr[pl.ds((dy * 6 + dxg) * 16, 16), pl.ds(j * 128, 128)] = t
        return 0

    lax.fori_loop(0, 33, asm, 0)
    r = sr[...]
    c = jnp.dot(w2_ref[...], r, preferred_element_type=jnp.float32)
    c = jnp.maximum(c + b2_ref[...], 0.0)
    sc2[...] = c

    def fill2(j, _):
        yo = j // 3
        g = j - yo * 3
        p = sc2[:, pl.ds(j * 128, 128)]
        s2[yo, pl.ds(g * 4, 4)] = p.reshape(4, 32, 128)
        return 0

    lax.fori_loop(0, 33, fill2, 0)

    pieces = []
    for yo2 in range(5):
        r0 = s2[2 * yo2]
        r1 = s2[2 * yo2 + 1]
        m = jnp.maximum(r0, r1)[0:10]
        p = jnp.max(m.reshape(5, 2, 32, 128), axis=1)
        pieces.append(p.reshape(160, 128))
    emb = jnp.concatenate(pieces, axis=0)

    t1 = jnp.tanh(jnp.dot(a1w_ref[...], emb,
                          preferred_element_type=jnp.float32) + a1b_ref[...])
    lg = jnp.dot(a1ow_ref[...], t1,
                 preferred_element_type=jnp.float32) + a1ob_ref[...]
    e = jnp.exp(jax.nn.sigmoid(lg))

    seg = lax.broadcasted_iota(jnp.int32, (NBAGS, 128), 0)
    ohw = jnp.where(lab_ref[0] == seg, e, 0.0)
    contrib = lax.dot_general(ohw, emb, (((1,), (1,)), ((), ())),
                              preferred_element_type=jnp.float32)

    @pl.when(step == 0)
    def _():
        u_ref[...] = jnp.zeros((NBAGS, 800), jnp.float32)
        sv_ref[...] = jnp.zeros((8, 128), jnp.float32)

    u_ref[...] += contrib
    sv_ref[0:1, :] += e


def _k3_body(u_ref, sv_ref, a2w_ref, a2b_ref, a2ow_ref, a2ob_ref,
             cw_ref, cb_ref, cow_ref, cob_ref, o_ref):
    u = u_ref[...]
    inv_s = 1.0 / jnp.sum(sv_ref[...])
    t3 = jnp.tanh(jnp.dot(u, a2w_ref[...],
                          preferred_element_type=jnp.float32) * inv_s
                  + a2b_ref[...])
    l3 = jax.nn.sigmoid(jnp.dot(t3, a2ow_ref[...],
                                preferred_element_type=jnp.float32)
                        + a2ob_ref[...])
    w3 = jnp.exp(l3)
    z = jnp.sum(w3)
    outer = lax.dot_general(w3, u, (((0,), (0,)), ((), ())),
                            preferred_element_type=jnp.float32)
    outer = outer * (inv_s / z)
    p1 = jnp.dot(outer, cw_ref[...],
                 preferred_element_type=jnp.float32) + cb_ref[...]
    p2 = jnp.dot(p1, cow_ref[...],
                 preferred_element_type=jnp.float32) + cob_ref[...]
    o_ref[...] = jnp.broadcast_to(jax.nn.sigmoid(p2), (8, 128))


def kernel(x, first_lab, conv1_w, conv1_b, conv2_w, conv2_b, a1_w, a1_b,
           a1o_w, a1o_b, a2_w, a2_b, a2o_w, a2o_b, c_w, c_b, co_w, co_b):
    xt = jnp.transpose(x.reshape(NBLK, BLK, 28, 28), (0, 2, 3, 1))
    t1 = conv1_w[:, :, 0, :]
    w1s = jnp.stack(
        [jnp.transpose(jnp.pad(t1, ((0, 0), (xo, 29 - xo), (0, 0))),
                       (2, 0, 1)).reshape(16, 96)
         for xo in range(26)], axis=0).reshape(416, 96)
    b1s = jnp.tile(conv1_b.reshape(1, 16), (26, 1)).reshape(416, 1)
    t = jnp.transpose(conv2_w, (3, 0, 1, 2))
    w2big = jnp.stack(
        [jnp.pad(t, ((0, 0), (0, 0), (xi, 3 - xi), (0, 0))).reshape(32, 288)
         for xi in range(4)], axis=0).reshape(128, 288)
    b2big = jnp.tile(conv2_b.reshape(1, 32), (4, 1)).reshape(128, 1)
    a1wt = a1_w.T
    a1bc = a1_b.reshape(64, 1)
    a1owt = a1o_w.reshape(1, 64)
    a1obc = a1o_b.reshape(1, 1)
    lab3 = first_lab.astype(jnp.int32).reshape(NBLK, 1, BLK)

    full = lambda shape: pl.BlockSpec(shape, lambda i: tuple(0 for _ in shape))
    u, sv = pl.pallas_call(
        _k1_body,
        grid=(NBLK,),
        in_specs=[
            pl.BlockSpec((1, 28, 28, BLK), lambda i: (i, 0, 0, 0)),
            full((416, 96)),
            full((416, 1)),
            full((128, 288)),
            full((128, 1)),
            full((64, 800)),
            full((64, 1)),
            full((1, 64)),
            full((1, 1)),
            pl.BlockSpec((1, 1, BLK), lambda i: (i, 0, 0)),
        ],
        out_specs=[full((NBAGS, 800)), full((8, 128))],
        out_shape=[jax.ShapeDtypeStruct((NBAGS, 800), jnp.float32),
                   jax.ShapeDtypeStruct((8, 128), jnp.float32)],
        scratch_shapes=[
            pltpu.VMEM((13, 14, 16, BLK), jnp.float32),
            pltpu.VMEM((288, 33 * BLK), jnp.float32),
            pltpu.VMEM((128, 33 * BLK), jnp.float32),
            pltpu.VMEM((11, 12, 32, BLK), jnp.float32),
            pltpu.VMEM((96, 26 * BLK), jnp.float32),
            pltpu.VMEM((416, 26 * BLK), jnp.float32),
        ],
    )(xt, w1s, b1s, w2big, b2big, a1wt, a1bc, a1owt, a1obc, lab3)

    out = pl.pallas_call(
        _k3_body,
        out_shape=jax.ShapeDtypeStruct((8, 128), jnp.float32),
    )(u, sv, a2_w, a2_b.reshape(1, 64), a2o_w, a2o_b.reshape(1, 1),
      c_w, c_b.reshape(1, 128), co_w, co_b.reshape(1, 1))
    return out[0:1, 0:1]
